# Initial kernel scaffold; baseline (speedup 1.0000x reference)
import jax
import jax.numpy as jnp
from jax.experimental import pallas as pl


def kernel(cat, cont, tables, W1, b1, W2, b2, Wout, bout):
    raise NotImplementedError("write your pallas kernel here")


# speedup vs baseline: 1.3053x; 1.3053x over previous
import functools

import jax
import jax.numpy as jnp
from jax import lax
from jax.experimental import pallas as pl
from jax.experimental.pallas import tpu as pltpu
from jax.experimental.pallas import tpu_sc as plsc


def _make_sc_gather(n_rows, d, nw, n_chunks):
    b_per_w = n_rows // nw
    chunk = b_per_w // n_chunks
    mesh = plsc.VectorSubcoreMesh(core_axis_name="c", subcore_axis_name="s")

    @functools.partial(
        pl.kernel,
        mesh=mesh,
        compiler_params=pltpu.CompilerParams(use_tc_tiling_on_sc=False),
        out_type=jax.ShapeDtypeStruct((n_rows, d), jnp.float32),
        scratch_types=[
            pltpu.VMEM((b_per_w,), jnp.int32),
            pltpu.VMEM((chunk, d), jnp.float32),
            pltpu.VMEM((chunk, d), jnp.float32),
            pltpu.SemaphoreType.DMA,
            pltpu.SemaphoreType.DMA,
            pltpu.SemaphoreType.DMA,
            pltpu.SemaphoreType.DMA,
        ],
    )
    def gather_kernel(table_hbm, idx_hbm, out_hbm, idx_v, buf0, buf1,
                      gsem0, gsem1, psem0, psem1):
        wid = lax.axis_index("s") * 2 + lax.axis_index("c")
        base = wid * b_per_w
        pltpu.sync_copy(idx_hbm.at[pl.ds(base, b_per_w)], idx_v)

        bufs = (buf0, buf1)
        gsems = (gsem0, gsem1)
        psems = (psem0, psem1)
        gathers = [None, None]
        puts = [None, None]
        for c in range(n_chunks):
            j = c & 1
            if puts[j] is not None:
                puts[j].wait()
            gathers[j] = pltpu.async_copy(
                table_hbm.at[idx_v.at[pl.ds(c * chunk, chunk)]],
                bufs[j], gsems[j])
            gathers[j].wait()
            puts[j] = pltpu.async_copy(
                bufs[j], out_hbm.at[pl.ds(base + c * chunk, chunk)], psems[j])
        for j in range(2):
            if puts[j] is not None:
                puts[j].wait()

    return gather_kernel


def _mlp_body(xc_ref, ct_ref, w1_ref, b1_ref, w2_ref, b2_ref,
              wa_ref, wb_ref, bo_ref, o_ref):
    h_cat = jnp.dot(xc_ref[...], w1_ref[...],
                    preferred_element_type=jnp.float32) + b1_ref[...]
    h_cont = jnp.dot(ct_ref[...], w2_ref[...],
                     preferred_element_type=jnp.float32) + b2_ref[...]
    h_cat = jnp.maximum(h_cat, 0.0)
    h_cont = jnp.maximum(h_cont, 0.0)
    o_ref[...] = (jnp.dot(h_cont, wa_ref[...],
                          preferred_element_type=jnp.float32)
                  + jnp.dot(h_cat, wb_ref[...],
                            preferred_element_type=jnp.float32)
                  + bo_ref[...])


def _mlp(x_cat, cont, W1, b1, W2, b2, Wout, bout, block_b=2048):
    n_b, k_cat = x_cat.shape
    c_dim = cont.shape[1]
    h_dim = W1.shape[1]
    out_dim = Wout.shape[1]
    wout_a = Wout[:h_dim]
    wout_b = Wout[h_dim:]
    b1_2d = b1.reshape(1, h_dim)
    b2_2d = b2.reshape(1, h_dim)
    bout_2d = bout.reshape(1, out_dim)
    grid = (n_b // block_b,)

    def full(shape):
        return pl.BlockSpec(shape, lambda i: (0, 0))

    return pl.pallas_call(
        _mlp_body,
        grid=grid,
        in_specs=[
            pl.BlockSpec((block_b, k_cat), lambda i: (i, 0)),
            pl.BlockSpec((block_b, c_dim), lambda i: (i, 0)),
            full(W1.shape),
            full(b1_2d.shape),
            full(W2.shape),
            full(b2_2d.shape),
            full(wout_a.shape),
            full(wout_b.shape),
            full(bout_2d.shape),
        ],
        out_specs=pl.BlockSpec((block_b, out_dim), lambda i: (i, 0)),
        out_shape=jax.ShapeDtypeStruct((n_b, out_dim), jnp.float32),
    )(x_cat, cont, W1, b1_2d, W2, b2_2d, wout_a, wout_b, bout_2d)


def kernel(cat, cont, tables, W1, b1, W2, b2, Wout, bout):
    n_f, v, d = tables.shape
    n_b = cat.shape[0]
    table_flat = tables.reshape(n_f * v, d)
    idx = (cat + jnp.arange(n_f, dtype=jnp.int32)[None, :] * v).reshape(-1)

    info = plsc.get_sparse_core_info()
    nw = info.num_cores * info.num_subcores
    n_rows = n_b * n_f
    gather = _make_sc_gather(n_rows, d, nw, n_chunks=8)
    emb_flat = gather(table_flat, idx)

    x_cat = emb_flat.reshape(n_b, n_f * d)
    return _mlp(x_cat, cont, W1, b1, W2, b2, Wout, bout)


# speedup vs baseline: 3.8601x; 2.9572x over previous
import functools

import jax
import jax.numpy as jnp
from jax import lax
from jax.experimental import pallas as pl
from jax.experimental.pallas import tpu as pltpu
from jax.experimental.pallas import tpu_sc as plsc


def _make_sc_colgather(n_f, d, v, n_b, nw):
    rows_per_w = (n_f * d) // nw
    mesh = plsc.VectorSubcoreMesh(core_axis_name="c", subcore_axis_name="s")

    @functools.partial(
        pl.kernel,
        mesh=mesh,
        compiler_params=pltpu.CompilerParams(use_tc_tiling_on_sc=False),
        out_type=jax.ShapeDtypeStruct((n_f * d, n_b), jnp.float32),
        scratch_types=[
            pltpu.VMEM((n_b,), jnp.int32),
            pltpu.VMEM((n_b,), jnp.int32),
            pltpu.VMEM((n_b,), jnp.float32),
            pltpu.VMEM((n_b,), jnp.float32),
            pltpu.SemaphoreType.DMA,
            pltpu.SemaphoreType.DMA,
            pltpu.SemaphoreType.DMA,
            pltpu.SemaphoreType.DMA,
            pltpu.SemaphoreType.DMA,
            pltpu.SemaphoreType.DMA,
        ],
    )
    def gather_kernel(tbl_hbm, catt_hbm, out_hbm, idx0, idx1, buf0, buf1,
                      isem0, isem1, gsem0, gsem1, psem0, psem1):
        wid = lax.axis_index("s") * 2 + lax.axis_index("c")
        base = wid * rows_per_w
        idxs = (idx0, idx1)
        bufs = (buf0, buf1)
        isems = (isem0, isem1)
        gsems = (gsem0, gsem1)
        psems = (psem0, psem1)

        def idx_load(r, j):
            fd = base + r
            return pltpu.async_copy(catt_hbm.at[fd // d], idxs[j], isems[j])

        icopies = [None, None]
        gcopies = [None, None]
        pcopies = [None, None]
        icopies[0] = idx_load(0, 0)
        for r in range(rows_per_w):
            j = r & 1
            fd = base + r
            if r + 1 < rows_per_w:
                icopies[1 - j] = idx_load(r + 1, 1 - j)
            icopies[j].wait()
            if pcopies[j] is not None:
                pcopies[j].wait()
            gcopies[j] = pltpu.async_copy(
                tbl_hbm.at[fd // d, fd % d].at[idxs[j]], bufs[j], gsems[j])
            gcopies[j].wait()
            pcopies[j] = pltpu.async_copy(bufs[j], out_hbm.at[fd], psems[j])
        for j in range(2):
            if pcopies[j] is not None:
                pcopies[j].wait()

    return gather_kernel


def _mlp_t_body(xc_ref, ct_ref, w1t_ref, b1_ref, w2t_ref, b2_ref,
                wat_ref, wbt_ref, bo_ref, o_ref):
    h_cat = jnp.dot(w1t_ref[...], xc_ref[...],
                    preferred_element_type=jnp.float32) + b1_ref[...]
    h_cont = jnp.dot(w2t_ref[...], ct_ref[...],
                     preferred_element_type=jnp.float32) + b2_ref[...]
    h_cat = jnp.maximum(h_cat, 0.0)
    h_cont = jnp.maximum(h_cont, 0.0)
    o_ref[...] = (jnp.dot(wat_ref[...], h_cont,
                          preferred_element_type=jnp.float32)
                  + jnp.dot(wbt_ref[...], h_cat,
                            preferred_element_type=jnp.float32)
                  + bo_ref[...])


def _mlp_t(x_catt, cont_t, W1, b1, W2, b2, Wout, bout, block_b=2048):
    k_cat, n_b = x_catt.shape
    c_dim = cont_t.shape[0]
    h_dim = W1.shape[1]
    out_dim = Wout.shape[1]
    w1t = W1.T
    w2t = W2.T
    wout_at = Wout[:h_dim].T
    wout_bt = Wout[h_dim:].T
    b1_2d = b1.reshape(h_dim, 1)
    b2_2d = b2.reshape(h_dim, 1)
    bout_2d = bout.reshape(out_dim, 1)
    grid = (n_b // block_b,)

    def full(shape):
        return pl.BlockSpec(shape, lambda i: (0, 0))

    out_t = pl.pallas_call(
        _mlp_t_body,
        grid=grid,
        in_specs=[
            pl.BlockSpec((k_cat, block_b), lambda i: (0, i)),
            pl.BlockSpec((c_dim, block_b), lambda i: (0, i)),
            full(w1t.shape),
            full(b1_2d.shape),
            full(w2t.shape),
            full(b2_2d.shape),
            full(wout_at.shape),
            full(wout_bt.shape),
            full(bout_2d.shape),
        ],
        out_specs=pl.BlockSpec((out_dim, block_b), lambda i: (0, i)),
        out_shape=jax.ShapeDtypeStruct((out_dim, n_b), jnp.float32),
    )(x_catt, cont_t, w1t, b1_2d, w2t, b2_2d, wout_at, wout_bt, bout_2d)
    return out_t.T


def kernel(cat, cont, tables, W1, b1, W2, b2, Wout, bout):
    n_f, v, d = tables.shape
    n_b = cat.shape[0]
    tables_t = jnp.transpose(tables, (0, 2, 1))
    cat_t = cat.T
    cont_t = cont.T

    info = plsc.get_sparse_core_info()
    nw = info.num_cores * info.num_subcores
    gather = _make_sc_colgather(n_f, d, v, n_b, nw)
    x_catt = gather(tables_t, cat_t)

    return _mlp_t(x_catt, cont_t, W1, b1, W2, b2, Wout, bout)


# speedup vs baseline: 20.4060x; 5.2864x over previous
import functools

import jax
import jax.numpy as jnp
from jax import lax
from jax.experimental import pallas as pl
from jax.experimental.pallas import tpu as pltpu
from jax.experimental.pallas import tpu_sc as plsc


_VP = 100096


def _detile_body(in_ref, out_ref):
    v = in_ref.shape[2]
    for dd in range(8):
        out_ref[pl.ds(dd * _VP, v)] = in_ref[0, dd, :]


def _detile(tables_t):
    n_f, d, v = tables_t.shape
    grid = (n_f, d // 8)
    return pl.pallas_call(
        _detile_body,
        grid=grid,
        in_specs=[pl.BlockSpec((1, 8, v), lambda f, g: (f, g, 0))],
        out_specs=pl.BlockSpec((8 * _VP,), lambda f, g: (f * (d // 8) + g,)),
        out_shape=jax.ShapeDtypeStruct((n_f * d * _VP,), jnp.float32),
    )(tables_t)


def _make_sc_colgather(n_f, d, v, n_b, nw):
    rows_per_w = (n_f * d) // nw
    mesh = plsc.VectorSubcoreMesh(core_axis_name="c", subcore_axis_name="s")

    @functools.partial(
        pl.kernel,
        mesh=mesh,
        compiler_params=pltpu.CompilerParams(use_tc_tiling_on_sc=False),
        out_type=jax.ShapeDtypeStruct((n_f * d, n_b), jnp.float32),
        scratch_types=[
            pltpu.VMEM((n_b,), jnp.int32),
            pltpu.VMEM((n_b,), jnp.int32),
            pltpu.VMEM((n_b,), jnp.float32),
            pltpu.VMEM((n_b,), jnp.float32),
            pltpu.SemaphoreType.DMA,
            pltpu.SemaphoreType.DMA,
            pltpu.SemaphoreType.DMA,
            pltpu.SemaphoreType.DMA,
            pltpu.SemaphoreType.DMA,
            pltpu.SemaphoreType.DMA,
        ],
    )
    def gather_kernel(tbl_hbm, catt_hbm, out_hbm, idx0, idx1, buf0, buf1,
                      isem0, isem1, gsem0, gsem1, psem0, psem1):
        wid = lax.axis_index("s") * 2 + lax.axis_index("c")
        base = wid * rows_per_w
        idxs = (idx0, idx1)
        bufs = (buf0, buf1)
        isems = (isem0, isem1)
        gsems = (gsem0, gsem1)
        psems = (psem0, psem1)

        def idx_load(r, j):
            fd = base + r
            return pltpu.async_copy(catt_hbm.at[fd // d], idxs[j], isems[j])

        icopies = [None, None]
        gcopies = [None, None]
        pcopies = [None, None]
        icopies[0] = idx_load(0, 0)
        for r in range(rows_per_w):
            j = r & 1
            fd = base + r
            if r + 1 < rows_per_w:
                icopies[1 - j] = idx_load(r + 1, 1 - j)
            icopies[j].wait()
            if pcopies[j] is not None:
                pcopies[j].wait()
            gcopies[j] = pltpu.async_copy(
                tbl_hbm.at[fd].at[idxs[j]], bufs[j], gsems[j])
            gcopies[j].wait()
            pcopies[j] = pltpu.async_copy(bufs[j], out_hbm.at[fd], psems[j])
        for j in range(2):
            if pcopies[j] is not None:
                pcopies[j].wait()

    return gather_kernel


def _mlp_t_body(xc_ref, ct_ref, w1t_ref, b1_ref, w2t_ref, b2_ref,
                wat_ref, wbt_ref, bo_ref, o_ref):
    h_cat = jnp.dot(w1t_ref[...], xc_ref[...],
                    preferred_element_type=jnp.float32) + b1_ref[...]
    h_cont = jnp.dot(w2t_ref[...], ct_ref[...],
                     preferred_element_type=jnp.float32) + b2_ref[...]
    h_cat = jnp.maximum(h_cat, 0.0)
    h_cont = jnp.maximum(h_cont, 0.0)
    o_ref[...] = (jnp.dot(wat_ref[...], h_cont,
                          preferred_element_type=jnp.float32)
                  + jnp.dot(wbt_ref[...], h_cat,
                            preferred_element_type=jnp.float32)
                  + bo_ref[...])


def _mlp_t(x_catt, cont_t, W1, b1, W2, b2, Wout, bout, block_b=2048):
    k_cat, n_b = x_catt.shape
    c_dim = cont_t.shape[0]
    h_dim = W1.shape[1]
    out_dim = Wout.shape[1]
    w1t = W1.T
    w2t = W2.T
    wout_at = Wout[:h_dim].T
    wout_bt = Wout[h_dim:].T
    b1_2d = b1.reshape(h_dim, 1)
    b2_2d = b2.reshape(h_dim, 1)
    bout_2d = bout.reshape(out_dim, 1)
    grid = (n_b // block_b,)

    def full(shape):
        return pl.BlockSpec(shape, lambda i: (0, 0))

    out_t = pl.pallas_call(
        _mlp_t_body,
        grid=grid,
        in_specs=[
            pl.BlockSpec((k_cat, block_b), lambda i: (0, i)),
            pl.BlockSpec((c_dim, block_b), lambda i: (0, i)),
            full(w1t.shape),
            full(b1_2d.shape),
            full(w2t.shape),
            full(b2_2d.shape),
            full(wout_at.shape),
            full(wout_bt.shape),
            full(bout_2d.shape),
        ],
        out_specs=pl.BlockSpec((out_dim, block_b), lambda i: (0, i)),
        out_shape=jax.ShapeDtypeStruct((out_dim, n_b), jnp.float32),
    )(x_catt, cont_t, w1t, b1_2d, w2t, b2_2d, wout_at, wout_bt, bout_2d)
    return out_t.T


def kernel(cat, cont, tables, W1, b1, W2, b2, Wout, bout):
    n_f, v, d = tables.shape
    n_b = cat.shape[0]
    tables_t = jnp.transpose(tables, (0, 2, 1))
    cat_t = cat.T
    cont_t = cont.T

    scratch = _detile(tables_t).reshape(n_f * d, _VP)

    info = plsc.get_sparse_core_info()
    nw = info.num_cores * info.num_subcores
    gather = _make_sc_colgather(n_f, d, v, n_b, nw)
    x_catt = gather(scratch, cat_t)

    return _mlp_t(x_catt, cont_t, W1, b1, W2, b2, Wout, bout)


# speedup vs baseline: 21.3655x; 1.0470x over previous
import functools

import jax
import jax.numpy as jnp
from jax import lax
from jax.experimental import pallas as pl
from jax.experimental.pallas import tpu as pltpu
from jax.experimental.pallas import tpu_sc as plsc

_VP = 100096
_SPLIT = (2, 8, 16)


def _detile_body(in_ref, out_ref):
    v = in_ref.shape[2]
    for dd in range(8):
        out_ref[pl.ds(dd * _VP, v)] = in_ref[0, dd, :]


def _detile(tables_t, f0, n_f):
    d, v = tables_t.shape[1], tables_t.shape[2]
    grid = (n_f, d // 8)
    return pl.pallas_call(
        _detile_body,
        grid=grid,
        in_specs=[pl.BlockSpec((1, 8, v), lambda f, g: (f + f0, g, 0))],
        out_specs=pl.BlockSpec((8 * _VP,), lambda f, g: (f * (d // 8) + g,)),
        out_shape=jax.ShapeDtypeStruct((n_f * d * _VP,), jnp.float32),
    )(tables_t)


def _make_sc_colgather(f0, n_f, d, n_b, nw):
    rows = n_f * d
    rows_per_w = rows // nw
    mesh = plsc.VectorSubcoreMesh(core_axis_name="c", subcore_axis_name="s")

    @functools.partial(
        pl.kernel,
        mesh=mesh,
        compiler_params=pltpu.CompilerParams(use_tc_tiling_on_sc=False),
        out_type=jax.ShapeDtypeStruct((rows, n_b), jnp.float32),
        scratch_types=[
            pltpu.VMEM((n_b,), jnp.int32),
            pltpu.VMEM((n_b,), jnp.int32),
            pltpu.VMEM((n_b,), jnp.float32),
            pltpu.VMEM((n_b,), jnp.float32),
            pltpu.SemaphoreType.DMA,
            pltpu.SemaphoreType.DMA,
            pltpu.SemaphoreType.DMA,
            pltpu.SemaphoreType.DMA,
            pltpu.SemaphoreType.DMA,
            pltpu.SemaphoreType.DMA,
        ],
    )
    def gather_kernel(tbl_hbm, catt_hbm, out_hbm, idx0, idx1, buf0, buf1,
                      isem0, isem1, gsem0, gsem1, psem0, psem1):
        wid = lax.axis_index("s") * 2 + lax.axis_index("c")
        base = wid * rows_per_w
        idxs = (idx0, idx1)
        bufs = (buf0, buf1)
        isems = (isem0, isem1)
        gsems = (gsem0, gsem1)
        psems = (psem0, psem1)

        def idx_load(r, j):
            fd = base + r
            return pltpu.async_copy(catt_hbm.at[f0 + fd // d], idxs[j],
                                    isems[j])

        icopies = [None, None]
        gcopies = [None, None]
        pcopies = [None, None]
        icopies[0] = idx_load(0, 0)
        for r in range(rows_per_w):
            j = r & 1
            fd = base + r
            if r + 1 < rows_per_w:
                icopies[1 - j] = idx_load(r + 1, 1 - j)
            icopies[j].wait()
            if pcopies[j] is not None:
                pcopies[j].wait()
            gcopies[j] = pltpu.async_copy(
                tbl_hbm.at[fd].at[idxs[j]], bufs[j], gsems[j])
            gcopies[j].wait()
            pcopies[j] = pltpu.async_copy(bufs[j], out_hbm.at[fd], psems[j])
        for j in range(2):
            if pcopies[j] is not None:
                pcopies[j].wait()

    return gather_kernel


def _mlp_t_body(ct_ref, w2t_ref, b2_ref, wat_ref, wbt_ref, bo_ref, b1_ref,
                *refs):
    n_pieces = (len(refs) - 1) // 2
    xc_refs = refs[:n_pieces]
    w1t_refs = refs[n_pieces:2 * n_pieces]
    o_ref = refs[-1]
    h_cat = b1_ref[...]
    for xc, w1t in zip(xc_refs, w1t_refs):
        h_cat = h_cat + jnp.dot(w1t[...], xc[...],
                                preferred_element_type=jnp.float32)
    h_cont = jnp.dot(w2t_ref[...], ct_ref[...],
                     preferred_element_type=jnp.float32) + b2_ref[...]
    h_cat = jnp.maximum(h_cat, 0.0)
    h_cont = jnp.maximum(h_cont, 0.0)
    o_ref[...] = (jnp.dot(wat_ref[...], h_cont,
                          preferred_element_type=jnp.float32)
                  + jnp.dot(wbt_ref[...], h_cat,
                            preferred_element_type=jnp.float32)
                  + bo_ref[...])


def _mlp_t(xc_pieces, cont_t, W1, b1, W2, b2, Wout, bout, block_b=2048):
    n_b = cont_t.shape[1]
    c_dim = cont_t.shape[0]
    h_dim = W1.shape[1]
    out_dim = Wout.shape[1]
    w1t = W1.T
    w2t = W2.T
    wout_at = Wout[:h_dim].T
    wout_bt = Wout[h_dim:].T
    b1_2d = b1.reshape(h_dim, 1)
    b2_2d = b2.reshape(h_dim, 1)
    bout_2d = bout.reshape(out_dim, 1)
    w1t_pieces = []
    col = 0
    for xc in xc_pieces:
        w1t_pieces.append(w1t[:, col:col + xc.shape[0]])
        col += xc.shape[0]
    grid = (n_b // block_b,)

    def full(shape):
        return pl.BlockSpec(shape, lambda i: (0, 0))

    in_specs = [
        pl.BlockSpec((c_dim, block_b), lambda i: (0, i)),
        full(w2t.shape),
        full(b2_2d.shape),
        full(wout_at.shape),
        full(wout_bt.shape),
        full(bout_2d.shape),
        full(b1_2d.shape),
    ]
    for xc in xc_pieces:
        in_specs.append(pl.BlockSpec((xc.shape[0], block_b), lambda i: (0, i)))
    for w in w1t_pieces:
        in_specs.append(full(w.shape))

    out_t = pl.pallas_call(
        _mlp_t_body,
        grid=grid,
        in_specs=in_specs,
        out_specs=pl.BlockSpec((out_dim, block_b), lambda i: (0, i)),
        out_shape=jax.ShapeDtypeStruct((out_dim, n_b), jnp.float32),
    )(cont_t, w2t, b2_2d, wout_at, wout_bt, bout_2d, b1_2d,
      *xc_pieces, *w1t_pieces)
    return out_t.T


def kernel(cat, cont, tables, W1, b1, W2, b2, Wout, bout):
    n_f, v, d = tables.shape
    n_b = cat.shape[0]
    tables_t = jnp.transpose(tables, (0, 2, 1))
    cat_t = cat.T
    cont_t = cont.T

    info = plsc.get_sparse_core_info()
    nw = info.num_cores * info.num_subcores

    xc_pieces = []
    f0 = 0
    for nf in _SPLIT:
        scratch = _detile(tables_t, f0, nf).reshape(nf * d, _VP)
        gather = _make_sc_colgather(f0, nf, d, n_b, nw)
        xc_pieces.append(gather(scratch, cat_t))
        f0 += nf

    return _mlp_t(xc_pieces, cont_t, W1, b1, W2, b2, Wout, bout)


# speedup vs baseline: 33.7437x; 1.5794x over previous
import functools

import jax
import jax.numpy as jnp
from jax import lax
from jax.experimental import pallas as pl
from jax.experimental.pallas import tpu as pltpu
from jax.experimental.pallas import tpu_sc as plsc

_VP = 100096
_SPLIT = (2, 8, 16)


def _detile_body(in_ref, out_ref):
    v = in_ref.shape[2]
    for dd in range(8):
        out_ref[pl.ds(dd * _VP, v)] = in_ref[0, dd, :]


def _detile(tables_t, f0, n_f):
    d, v = tables_t.shape[1], tables_t.shape[2]
    grid = (n_f, d // 8)
    return pl.pallas_call(
        _detile_body,
        grid=grid,
        in_specs=[pl.BlockSpec((1, 8, v), lambda f, g: (f + f0, g, 0))],
        out_specs=pl.BlockSpec((8 * _VP,), lambda f, g: (f * (d // 8) + g,)),
        out_shape=jax.ShapeDtypeStruct((n_f * d * _VP,), jnp.float32),
    )(tables_t)


def _make_sc_colgather(f0, n_f, d, n_b, nw):
    nf2 = n_f // 2
    qb = n_b // 4
    mesh = plsc.VectorSubcoreMesh(core_axis_name="c", subcore_axis_name="s")

    @functools.partial(
        pl.kernel,
        mesh=mesh,
        compiler_params=pltpu.CompilerParams(use_tc_tiling_on_sc=False),
        out_type=jax.ShapeDtypeStruct((n_f * d, n_b), jnp.float32),
        scratch_types=[
            pltpu.VMEM_SHARED((2, 4, _VP), jnp.float32),
            pltpu.VMEM((n_b // 4,), jnp.int32),
            pltpu.VMEM((n_b // 4,), jnp.int32),
            pltpu.VMEM((n_b // 4,), jnp.float32),
            pltpu.VMEM((n_b // 4,), jnp.float32),
            pltpu.SemaphoreType.DMA,
            pltpu.SemaphoreType.DMA,
            pltpu.SemaphoreType.DMA,
            pltpu.SemaphoreType.DMA,
            pltpu.SemaphoreType.DMA,
            pltpu.SemaphoreType.DMA,
            pltpu.SemaphoreType.DMA,
            pltpu.SemaphoreType.DMA,
        ],
    )
    def gather_kernel(tbl_hbm, catt_hbm, out_hbm, plane, idx0, idx1,
                      buf0, buf1, lsem0, lsem1, isem0, isem1,
                      gsem0, gsem1, psem0, psem1):
        c = lax.axis_index("c")
        sid = lax.axis_index("s")
        dd = sid % 4
        b0 = (sid // 4) * qb
        idxs = (idx0, idx1)
        bufs = (buf0, buf1)
        lsems = (lsem0, lsem1)
        isems = (isem0, isem1)
        gsems = (gsem0, gsem1)
        psems = (psem0, psem1)
        n_qp = nf2 * 4

        def plane_src(g):
            row0 = (c * nf2 + g // 4) * d + (g % 4) * 4
            return tbl_hbm.at[pl.ds(row0, 4)]

        def plane_issue(g, slot):
            @pl.when(sid == 0)
            def _():
                pltpu.async_copy(plane_src(g), plane.at[slot], lsems[slot])

        def plane_wait(g, slot):
            @pl.when(sid == 0)
            def _():
                pltpu.make_async_copy(plane_src(g), plane.at[slot],
                                      lsems[slot]).wait()

        def idx_load(k):
            f_loc = c * nf2 + k
            return pltpu.async_copy(
                catt_hbm.at[f0 + f_loc, pl.ds(b0, qb)], idxs[k & 1],
                isems[k & 1])

        pcopies = [None, None]
        plane_issue(0, 0)
        icopy = idx_load(0)
        for g in range(n_qp):
            slot = g & 1
            k = g // 4
            if g + 1 < n_qp:
                plane_issue(g + 1, 1 - slot)
            plane_wait(g, slot)
            if g % 4 == 0:
                icopy.wait()
            plsc.subcore_barrier()
            if pcopies[slot] is not None:
                pcopies[slot].wait()
            pltpu.async_copy(
                plane.at[slot, dd].at[idxs[k & 1]], bufs[slot],
                gsems[slot]).wait()
            row = (c * nf2 + k) * d + (g % 4) * 4 + dd
            pcopies[slot] = pltpu.async_copy(
                bufs[slot], out_hbm.at[row, pl.ds(b0, qb)], psems[slot])
            if g % 4 == 3 and k + 1 < nf2:
                icopy = idx_load(k + 1)
            plsc.subcore_barrier()
        for j in range(2):
            if pcopies[j] is not None:
                pcopies[j].wait()

    return gather_kernel


def _mlp_t_body(ct_ref, w2t_ref, b2_ref, wat_ref, wbt_ref, bo_ref, b1_ref,
                *refs):
    n_pieces = (len(refs) - 1) // 2
    xc_refs = refs[:n_pieces]
    w1t_refs = refs[n_pieces:2 * n_pieces]
    o_ref = refs[-1]
    h_cat = b1_ref[...]
    for xc, w1t in zip(xc_refs, w1t_refs):
        h_cat = h_cat + jnp.dot(w1t[...], xc[...],
                                preferred_element_type=jnp.float32)
    h_cont = jnp.dot(w2t_ref[...], ct_ref[...],
                     preferred_element_type=jnp.float32) + b2_ref[...]
    h_cat = jnp.maximum(h_cat, 0.0)
    h_cont = jnp.maximum(h_cont, 0.0)
    o_ref[...] = (jnp.dot(wat_ref[...], h_cont,
                          preferred_element_type=jnp.float32)
                  + jnp.dot(wbt_ref[...], h_cat,
                            preferred_element_type=jnp.float32)
                  + bo_ref[...])


def _mlp_t(xc_pieces, cont_t, W1, b1, W2, b2, Wout, bout, block_b=2048):
    n_b = cont_t.shape[1]
    c_dim = cont_t.shape[0]
    h_dim = W1.shape[1]
    out_dim = Wout.shape[1]
    w1t = W1.T
    w2t = W2.T
    wout_at = Wout[:h_dim].T
    wout_bt = Wout[h_dim:].T
    b1_2d = b1.reshape(h_dim, 1)
    b2_2d = b2.reshape(h_dim, 1)
    bout_2d = bout.reshape(out_dim, 1)
    w1t_pieces = []
    col = 0
    for xc in xc_pieces:
        w1t_pieces.append(w1t[:, col:col + xc.shape[0]])
        col += xc.shape[0]
    grid = (n_b // block_b,)

    def full(shape):
        return pl.BlockSpec(shape, lambda i: (0, 0))

    in_specs = [
        pl.BlockSpec((c_dim, block_b), lambda i: (0, i)),
        full(w2t.shape),
        full(b2_2d.shape),
        full(wout_at.shape),
        full(wout_bt.shape),
        full(bout_2d.shape),
        full(b1_2d.shape),
    ]
    for xc in xc_pieces:
        in_specs.append(pl.BlockSpec((xc.shape[0], block_b), lambda i: (0, i)))
    for w in w1t_pieces:
        in_specs.append(full(w.shape))

    out_t = pl.pallas_call(
        _mlp_t_body,
        grid=grid,
        in_specs=in_specs,
        out_specs=pl.BlockSpec((out_dim, block_b), lambda i: (0, i)),
        out_shape=jax.ShapeDtypeStruct((out_dim, n_b), jnp.float32),
    )(cont_t, w2t, b2_2d, wout_at, wout_bt, bout_2d, b1_2d,
      *xc_pieces, *w1t_pieces)
    return out_t.T


def kernel(cat, cont, tables, W1, b1, W2, b2, Wout, bout):
    n_f, v, d = tables.shape
    n_b = cat.shape[0]
    tables_t = jnp.transpose(tables, (0, 2, 1))
    cat_t = cat.T
    cont_t = cont.T

    info = plsc.get_sparse_core_info()
    nw = info.num_cores * info.num_subcores

    xc_pieces = []
    f0 = 0
    for nf in _SPLIT:
        scratch = _detile(tables_t, f0, nf).reshape(nf * d, _VP)
        gather = _make_sc_colgather(f0, nf, d, n_b, nw)
        xc_pieces.append(gather(scratch, cat_t))
        f0 += nf

    return _mlp_t(xc_pieces, cont_t, W1, b1, W2, b2, Wout, bout)
